# Initial kernel scaffold; baseline (speedup 1.0000x reference)
import jax
import jax.numpy as jnp
from jax.experimental import pallas as pl


def kernel(anchor, pos, W1, b1, W2, b2):
    raise NotImplementedError("write your pallas kernel here")


# speedup vs baseline: 1.2063x; 1.2063x over previous
import functools

import jax
import jax.numpy as jnp
from jax import lax
from jax.experimental import pallas as pl
from jax.experimental.pallas import tpu as pltpu
from jax.experimental.pallas import tpu_sc as plsc

B = 4096
D_IN = 512
D_HID = 1024
D_OUT = 128

BR = 1024
BA = 512

_PREC = lax.Precision.DEFAULT


def _encode_body(xa_ref, xp_ref, w1_ref, b1_ref, w2_ref, b2_ref, a_ref, p_ref):
    w1 = w1_ref[...]
    w2 = w2_ref[...]
    b1 = b1_ref[...]
    b2 = b2_ref[...]
    for x_ref, o_ref in ((xa_ref, a_ref), (xp_ref, p_ref)):
        h = jnp.dot(x_ref[...], w1, precision=_PREC,
                    preferred_element_type=jnp.float32)
        h = jnp.maximum(h + b1, 0.0)
        o_ref[...] = jnp.dot(h, w2, precision=_PREC,
                             preferred_element_type=jnp.float32) + b2


def _encode(anchor, pos, W1, b1, W2, b2):
    nb = B // BR
    return pl.pallas_call(
        _encode_body,
        grid=(nb,),
        in_specs=[
            pl.BlockSpec((BR, D_IN), lambda i: (i, 0)),
            pl.BlockSpec((BR, D_IN), lambda i: (i, 0)),
            pl.BlockSpec((D_IN, D_HID), lambda i: (0, 0)),
            pl.BlockSpec((1, D_HID), lambda i: (0, 0)),
            pl.BlockSpec((D_HID, D_OUT), lambda i: (0, 0)),
            pl.BlockSpec((1, D_OUT), lambda i: (0, 0)),
        ],
        out_specs=[
            pl.BlockSpec((BR, D_OUT), lambda i: (i, 0)),
            pl.BlockSpec((BR, D_OUT), lambda i: (i, 0)),
        ],
        out_shape=[
            jax.ShapeDtypeStruct((B, D_OUT), jnp.float32),
            jax.ShapeDtypeStruct((B, D_OUT), jnp.float32),
        ],
    )(anchor, pos, W1, b1.reshape(1, D_HID), W2, b2.reshape(1, D_OUT))


def _argmin_body(a_ref, p_ref, idx_ref):
    i = pl.program_id(0)
    a = a_ref[...]
    p = p_ref[...]
    an = jnp.sum(a * a, axis=1, keepdims=True)
    pn = jnp.sum(p * p, axis=1)[None, :]
    ap = lax.dot_general(a, p, (((1,), (1,)), ((), ())),
                         precision=_PREC, preferred_element_type=jnp.float32)
    dist = an + pn - 2.0 * ap
    rows = i * BA + lax.broadcasted_iota(jnp.int32, (BA, B), 0)
    cols = lax.broadcasted_iota(jnp.int32, (BA, B), 1)
    dist = jnp.where(rows == cols, dist + 1e20, dist)
    idx_ref[0, 0, :] = jnp.argmin(dist, axis=1).astype(jnp.int32)


def _argmin(a, p):
    nb = B // BA
    out = pl.pallas_call(
        _argmin_body,
        grid=(nb,),
        in_specs=[
            pl.BlockSpec((BA, D_OUT), lambda i: (i, 0)),
            pl.BlockSpec((B, D_OUT), lambda i: (0, 0)),
        ],
        out_specs=pl.BlockSpec((1, 1, BA), lambda i: (i, 0, 0)),
        out_shape=jax.ShapeDtypeStruct((nb, 1, BA), jnp.int32),
    )(a, p)
    return out.reshape(B)


def _sc_gather(table, idx):
    info = plsc.get_sparse_core_info()
    nc, ns = info.num_cores, info.num_subcores
    nw = nc * ns
    b_per_w = B // nw
    mesh = plsc.VectorSubcoreMesh(core_axis_name="c", subcore_axis_name="s")

    @functools.partial(
        pl.kernel,
        mesh=mesh,
        out_type=jax.ShapeDtypeStruct((B, D_OUT), jnp.float32),
        scratch_types=[
            pltpu.VMEM((b_per_w,), jnp.int32),
            pltpu.VMEM((b_per_w, D_OUT), jnp.float32),
            pltpu.SemaphoreType.DMA,
        ],
    )
    def gather_k(table_hbm, idx_hbm, out_hbm, idx_v, rows_v, sem):
        wid = lax.axis_index("s") * nc + lax.axis_index("c")
        base = wid * b_per_w
        pltpu.sync_copy(idx_hbm.at[pl.ds(base, b_per_w)], idx_v)
        pltpu.async_copy(table_hbm.at[idx_v], rows_v, sem).wait()
        pltpu.sync_copy(rows_v, out_hbm.at[pl.ds(base, b_per_w)])

    return gather_k(table, idx)


def kernel(anchor, pos, W1, b1, W2, b2):
    a, p = _encode(anchor, pos, W1, b1, W2, b2)
    idx = _argmin(a, p)
    neg = _sc_gather(p, idx)
    return jnp.concatenate([a, p, neg], axis=-1)


# speedup vs baseline: 1.2219x; 1.0129x over previous
import functools

import jax
import jax.numpy as jnp
from jax import lax
from jax.experimental import pallas as pl
from jax.experimental.pallas import tpu as pltpu
from jax.experimental.pallas import tpu_sc as plsc

B = 4096
D_IN = 512
D_HID = 1024
D_OUT = 128

BR = 1024
BA = 512

_PREC = lax.Precision.DEFAULT


def _encode_body(xa_ref, xp_ref, w1_ref, b1_ref, w2_ref, b2_ref, a_ref, p_ref):
    w1 = w1_ref[...]
    w2 = w2_ref[...]
    b1 = b1_ref[...]
    b2 = b2_ref[...]
    for x_ref, o_ref in ((xa_ref, a_ref), (xp_ref, p_ref)):
        h = jnp.dot(x_ref[...], w1, precision=_PREC,
                    preferred_element_type=jnp.float32)
        h = jnp.maximum(h + b1, 0.0)
        o_ref[...] = jnp.dot(h, w2, precision=_PREC,
                             preferred_element_type=jnp.float32) + b2


def _encode(anchor, pos, W1, b1, W2, b2):
    nb = B // BR
    return pl.pallas_call(
        _encode_body,
        grid=(nb,),
        in_specs=[
            pl.BlockSpec((BR, D_IN), lambda i: (i, 0)),
            pl.BlockSpec((BR, D_IN), lambda i: (i, 0)),
            pl.BlockSpec((D_IN, D_HID), lambda i: (0, 0)),
            pl.BlockSpec((1, D_HID), lambda i: (0, 0)),
            pl.BlockSpec((D_HID, D_OUT), lambda i: (0, 0)),
            pl.BlockSpec((1, D_OUT), lambda i: (0, 0)),
        ],
        out_specs=[
            pl.BlockSpec((BR, D_OUT), lambda i: (i, 0)),
            pl.BlockSpec((BR, D_OUT), lambda i: (i, 0)),
        ],
        out_shape=[
            jax.ShapeDtypeStruct((B, D_OUT), jnp.float32),
            jax.ShapeDtypeStruct((B, D_OUT), jnp.float32),
        ],
    )(anchor, pos, W1, b1.reshape(1, D_HID), W2, b2.reshape(1, D_OUT))


def _argmin_body(a_ref, p_ref, idx_ref):
    i = pl.program_id(0)
    a = a_ref[...]
    p = p_ref[...]
    an = jnp.sum(a * a, axis=1, keepdims=True)
    pn = jnp.sum(p * p, axis=1)[None, :]
    ap2 = lax.dot_general(a * (-2.0), p, (((1,), (1,)), ((), ())),
                          precision=_PREC, preferred_element_type=jnp.float32)
    dist = (an + pn) + ap2
    rows = i * BA + lax.broadcasted_iota(jnp.int32, (BA, B), 0)
    cols = lax.broadcasted_iota(jnp.int32, (BA, B), 1)
    dist = jnp.where(rows == cols, dist + 1e20, dist)
    idx_ref[0, 0, :] = jnp.argmin(dist, axis=1).astype(jnp.int32)


def _argmin(a, p):
    nb = B // BA
    out = pl.pallas_call(
        _argmin_body,
        grid=(nb,),
        in_specs=[
            pl.BlockSpec((BA, D_OUT), lambda i: (i, 0)),
            pl.BlockSpec((B, D_OUT), lambda i: (0, 0)),
        ],
        out_specs=pl.BlockSpec((1, 1, BA), lambda i: (i, 0, 0)),
        out_shape=jax.ShapeDtypeStruct((nb, 1, BA), jnp.int32),
    )(a, p)
    return out.reshape(B)


def _sc_finalize(a, p, idx):
    info = plsc.get_sparse_core_info()
    nc, ns = info.num_cores, info.num_subcores
    nw = nc * ns
    bw = B // nw
    mesh = plsc.VectorSubcoreMesh(core_axis_name="c", subcore_axis_name="s")

    @functools.partial(
        pl.kernel,
        mesh=mesh,
        out_type=jax.ShapeDtypeStruct((B, 3 * D_OUT), jnp.float32),
        scratch_types=[
            pltpu.VMEM((bw,), jnp.int32),
            pltpu.VMEM((bw, 3 * D_OUT), jnp.float32),
            pltpu.SemaphoreType.DMA,
        ],
    )
    def finalize_k(a_hbm, p_hbm, idx_hbm, out_hbm, idx_v, tile_v, sem):
        wid = lax.axis_index("s") * nc + lax.axis_index("c")
        base = wid * bw
        pltpu.sync_copy(idx_hbm.at[pl.ds(base, bw)], idx_v)
        pltpu.sync_copy(a_hbm.at[pl.ds(base, bw)], tile_v.at[:, pl.ds(0, D_OUT)])
        pltpu.sync_copy(p_hbm.at[pl.ds(base, bw)], tile_v.at[:, pl.ds(D_OUT, D_OUT)])
        pltpu.async_copy(p_hbm.at[idx_v], tile_v.at[:, pl.ds(2 * D_OUT, D_OUT)],
                         sem).wait()
        pltpu.sync_copy(tile_v, out_hbm.at[pl.ds(base, bw)])

    return finalize_k(a, p, idx)


def kernel(anchor, pos, W1, b1, W2, b2):
    a, p = _encode(anchor, pos, W1, b1, W2, b2)
    idx = _argmin(a, p)
    return _sc_finalize(a, p, idx)


# speedup vs baseline: 1.2338x; 1.0098x over previous
import functools

import jax
import jax.numpy as jnp
from jax import lax
from jax.experimental import pallas as pl
from jax.experimental.pallas import tpu as pltpu
from jax.experimental.pallas import tpu_sc as plsc

B = 4096
D_IN = 512
D_HID = 1024
D_OUT = 128

BR = 1024
BA = 512
NB_E = B // BR
NB_A = B // BA

_PREC = lax.Precision.DEFAULT


def _tc_body(xa_ref, xp_ref, w1_ref, b1_ref, w2_ref, b2_ref,
             a_out, p_out, idx_out, a_s, p_s, pn_s, dist_s):
    k = pl.program_id(0)

    @pl.when(k < NB_E)
    def _encode():
        w1 = w1_ref[...]
        b1 = b1_ref[...]
        w2 = w2_ref[...]
        b2 = b2_ref[...]
        base = k * BR
        for x_ref, o_ref, s_ref, is_p in ((xa_ref, a_out, a_s, False),
                                          (xp_ref, p_out, p_s, True)):
            h = jnp.maximum(jnp.dot(x_ref[...], w1, precision=_PREC,
                                    preferred_element_type=jnp.float32) + b1, 0.0)
            o = jnp.dot(h, w2, precision=_PREC,
                        preferred_element_type=jnp.float32) + b2
            o_ref[...] = o
            s_ref[pl.ds(base, BR), :] = o
            if is_p:
                pn_s[:, pl.ds(base, BR)] = jnp.sum(o * o, axis=1)[None, :]

    @pl.when(k >= NB_E)
    def _argmin():
        i = k - NB_E
        a = a_s[pl.ds(i * BA, BA), :]
        an = jnp.sum(a * a, axis=1, keepdims=True)
        ap2 = lax.dot_general(a * (-2.0), p_s[...], (((1,), (1,)), ((), ())),
                              precision=_PREC, preferred_element_type=jnp.float32)
        dist_s[...] = (an + pn_s[...]) + ap2
        sub = dist_s[:, pl.ds(i * BA, BA)]
        r = lax.broadcasted_iota(jnp.int32, (BA, BA), 0)
        c = lax.broadcasted_iota(jnp.int32, (BA, BA), 1)
        dist_s[:, pl.ds(i * BA, BA)] = jnp.where(r == c, sub + 1e20, sub)
        idx_out[0, 0, :] = jnp.argmin(dist_s[...], axis=1).astype(jnp.int32)


def _tc_encode_argmin(anchor, pos, W1, b1, W2, b2):
    return pl.pallas_call(
        _tc_body,
        grid=(NB_E + NB_A,),
        in_specs=[
            pl.BlockSpec((BR, D_IN), lambda k: (jnp.minimum(k, NB_E - 1), 0)),
            pl.BlockSpec((BR, D_IN), lambda k: (jnp.minimum(k, NB_E - 1), 0)),
            pl.BlockSpec((D_IN, D_HID), lambda k: (0, 0)),
            pl.BlockSpec((1, D_HID), lambda k: (0, 0)),
            pl.BlockSpec((D_HID, D_OUT), lambda k: (0, 0)),
            pl.BlockSpec((1, D_OUT), lambda k: (0, 0)),
        ],
        out_specs=[
            pl.BlockSpec((BR, D_OUT), lambda k: (jnp.minimum(k, NB_E - 1), 0)),
            pl.BlockSpec((BR, D_OUT), lambda k: (jnp.minimum(k, NB_E - 1), 0)),
            pl.BlockSpec((1, 1, BA), lambda k: (jnp.maximum(k - NB_E, 0), 0, 0)),
        ],
        out_shape=[
            jax.ShapeDtypeStruct((B, D_OUT), jnp.float32),
            jax.ShapeDtypeStruct((B, D_OUT), jnp.float32),
            jax.ShapeDtypeStruct((NB_A, 1, BA), jnp.int32),
        ],
        scratch_shapes=[
            pltpu.VMEM((B, D_OUT), jnp.float32),
            pltpu.VMEM((B, D_OUT), jnp.float32),
            pltpu.VMEM((1, B), jnp.float32),
            pltpu.VMEM((BA, B), jnp.float32),
        ],
    )(anchor, pos, W1, b1.reshape(1, D_HID), W2, b2.reshape(1, D_OUT))


def _sc_finalize(a, p, idx):
    info = plsc.get_sparse_core_info()
    nc, ns = info.num_cores, info.num_subcores
    nw = nc * ns
    bw = B // nw
    mesh = plsc.VectorSubcoreMesh(core_axis_name="c", subcore_axis_name="s")

    @functools.partial(
        pl.kernel,
        mesh=mesh,
        out_type=jax.ShapeDtypeStruct((B, 3 * D_OUT), jnp.float32),
        scratch_types=[
            pltpu.VMEM((bw,), jnp.int32),
            pltpu.VMEM((bw, 3 * D_OUT), jnp.float32),
            pltpu.SemaphoreType.DMA,
        ],
    )
    def finalize_k(a_hbm, p_hbm, idx_hbm, out_hbm, idx_v, tile_v, sem):
        wid = lax.axis_index("s") * nc + lax.axis_index("c")
        base = wid * bw
        pltpu.sync_copy(idx_hbm.at[pl.ds(base, bw)], idx_v)
        pltpu.sync_copy(a_hbm.at[pl.ds(base, bw)], tile_v.at[:, pl.ds(0, D_OUT)])
        pltpu.sync_copy(p_hbm.at[pl.ds(base, bw)], tile_v.at[:, pl.ds(D_OUT, D_OUT)])
        pltpu.async_copy(p_hbm.at[idx_v], tile_v.at[:, pl.ds(2 * D_OUT, D_OUT)],
                         sem).wait()
        pltpu.sync_copy(tile_v, out_hbm.at[pl.ds(base, bw)])

    return finalize_k(a, p, idx)


def kernel(anchor, pos, W1, b1, W2, b2):
    a, p, idx = _tc_encode_argmin(anchor, pos, W1, b1, W2, b2)
    return _sc_finalize(a, p, idx.reshape(B))


# speedup vs baseline: 1.2673x; 1.0271x over previous
import functools

import jax
import jax.numpy as jnp
from jax import lax
from jax.experimental import pallas as pl
from jax.experimental.pallas import tpu as pltpu
from jax.experimental.pallas import tpu_sc as plsc

B = 4096
D_IN = 512
D_HID = 1024
D_OUT = 128

BR = 1024
BA = 512
NB_E = B // BR
NB_A = B // BA

_PREC = lax.Precision.DEFAULT


def _tc_body(xa_ref, xp_ref, w1_ref, b1_ref, w2_ref, b2_ref,
             a_out, p_out, idx_out, a_s, p_s, pn_s, dist_s):
    k = pl.program_id(0)

    @pl.when(k < NB_E)
    def _encode():
        w1 = w1_ref[...]
        b1 = b1_ref[...]
        w2 = w2_ref[...]
        b2 = b2_ref[...]
        base = k * BR
        for x_ref, o_ref, s_ref, is_p in ((xa_ref, a_out, a_s, False),
                                          (xp_ref, p_out, p_s, True)):
            h = jnp.maximum(jnp.dot(x_ref[...], w1, precision=_PREC,
                                    preferred_element_type=jnp.float32) + b1, 0.0)
            o = jnp.dot(h, w2, precision=_PREC,
                        preferred_element_type=jnp.float32) + b2
            o_ref[...] = o
            s_ref[pl.ds(base, BR), :] = o
            if is_p:
                pn_s[:, pl.ds(base, BR)] = jnp.sum(o * o, axis=1)[None, :]

    @pl.when(k >= NB_E)
    def _argmin():
        i = k - NB_E
        a = a_s[pl.ds(i * BA, BA), :]
        an = jnp.sum(a * a, axis=1, keepdims=True)
        ap2 = lax.dot_general(a * (-2.0), p_s[...], (((1,), (1,)), ((), ())),
                              precision=_PREC, preferred_element_type=jnp.float32)
        dist = (an + pn_s[...]) + ap2
        rows = i * BA + lax.broadcasted_iota(jnp.int32, (BA, B), 0)
        cols = lax.broadcasted_iota(jnp.int32, (BA, B), 1)
        dist = jnp.where(rows == cols, dist + 1e20, dist)
        idx_out[0, 0, :] = jnp.argmin(dist, axis=1).astype(jnp.int32)


def _tc_encode_argmin(anchor, pos, W1, b1, W2, b2):
    return pl.pallas_call(
        _tc_body,
        grid=(NB_E + NB_A,),
        in_specs=[
            pl.BlockSpec((BR, D_IN), lambda k: (jnp.minimum(k, NB_E - 1), 0)),
            pl.BlockSpec((BR, D_IN), lambda k: (jnp.minimum(k, NB_E - 1), 0)),
            pl.BlockSpec((D_IN, D_HID), lambda k: (0, 0)),
            pl.BlockSpec((1, D_HID), lambda k: (0, 0)),
            pl.BlockSpec((D_HID, D_OUT), lambda k: (0, 0)),
            pl.BlockSpec((1, D_OUT), lambda k: (0, 0)),
        ],
        out_specs=[
            pl.BlockSpec((BR, D_OUT), lambda k: (jnp.minimum(k, NB_E - 1), 0)),
            pl.BlockSpec((BR, D_OUT), lambda k: (jnp.minimum(k, NB_E - 1), 0)),
            pl.BlockSpec((1, 1, BA), lambda k: (jnp.maximum(k - NB_E, 0), 0, 0)),
        ],
        out_shape=[
            jax.ShapeDtypeStruct((B, D_OUT), jnp.float32),
            jax.ShapeDtypeStruct((B, D_OUT), jnp.float32),
            jax.ShapeDtypeStruct((NB_A, 1, BA), jnp.int32),
        ],
        scratch_shapes=[
            pltpu.VMEM((B, D_OUT), jnp.float32),
            pltpu.VMEM((B, D_OUT), jnp.float32),
            pltpu.VMEM((1, B), jnp.float32),
            pltpu.VMEM((BA, B), jnp.float32),
        ],
    )(anchor, pos, W1, b1.reshape(1, D_HID), W2, b2.reshape(1, D_OUT))


def _sc_finalize(a, p, idx):
    info = plsc.get_sparse_core_info()
    nc, ns = info.num_cores, info.num_subcores
    nw = nc * ns
    bw = B // nw
    mesh = plsc.VectorSubcoreMesh(core_axis_name="c", subcore_axis_name="s")

    @functools.partial(
        pl.kernel,
        mesh=mesh,
        out_type=jax.ShapeDtypeStruct((B, 3 * D_OUT), jnp.float32),
        scratch_types=[
            pltpu.VMEM((bw,), jnp.int32),
            pltpu.VMEM((bw, 3 * D_OUT), jnp.float32),
            pltpu.SemaphoreType.DMA,
            pltpu.SemaphoreType.DMA,
            pltpu.SemaphoreType.DMA,
            pltpu.SemaphoreType.DMA,
        ],
    )
    def finalize_k(a_hbm, p_hbm, idx_hbm, out_hbm, idx_v, tile_v,
                   sem_i, sem_a, sem_p, sem_g):
        wid = lax.axis_index("s") * nc + lax.axis_index("c")
        base = wid * bw
        ci = pltpu.async_copy(idx_hbm.at[pl.ds(base, bw)], idx_v, sem_i)
        ca = pltpu.async_copy(a_hbm.at[pl.ds(base, bw)],
                              tile_v.at[:, pl.ds(0, D_OUT)], sem_a)
        cp = pltpu.async_copy(p_hbm.at[pl.ds(base, bw)],
                              tile_v.at[:, pl.ds(D_OUT, D_OUT)], sem_p)
        ci.wait()
        cg = pltpu.async_copy(p_hbm.at[idx_v],
                              tile_v.at[:, pl.ds(2 * D_OUT, D_OUT)], sem_g)
        ca.wait()
        cp.wait()
        cg.wait()
        pltpu.sync_copy(tile_v, out_hbm.at[pl.ds(base, bw)])

    return finalize_k(a, p, idx)


def kernel(anchor, pos, W1, b1, W2, b2):
    a, p, idx = _tc_encode_argmin(anchor, pos, W1, b1, W2, b2)
    return _sc_finalize(a, p, idx.reshape(B))
